# Initial kernel scaffold; baseline (speedup 1.0000x reference)
import jax
import jax.numpy as jnp
from jax.experimental import pallas as pl


def kernel(x, edge_index, W1, b1, g1, be1, W2, b2, g2, be2, W3, b3, g3, be3, W4, b4, g4, be4):
    raise NotImplementedError("write your pallas kernel here")


# speedup vs baseline: 17.9277x; 17.9277x over previous
import functools

import jax
import jax.numpy as jnp
from jax import lax
from jax.experimental import pallas as pl
from jax.experimental.pallas import tpu as pltpu
from jax.experimental.pallas import tpu_sc as plsc

N = 10000
E = 320000
D = 128
NP = 10240
NC = 2
NS = 16
NW = NC * NS
CK = 128
CH = 79
EW = CH * CK
EPAD = NW * EW
RPT = NP // NS
EPS = 1e-5

@functools.cache
def _sc_degree_kernel():
  mesh = plsc.VectorSubcoreMesh(core_axis_name="c", subcore_axis_name="s")

  @functools.partial(
      pl.kernel,
      out_type=jax.ShapeDtypeStruct((NC, NP), jnp.float32),
      mesh=mesh,
      scratch_types=[
          pltpu.VMEM_SHARED((NP,), jnp.float32),
          pltpu.VMEM((CH, CK), jnp.int32),
          pltpu.VMEM((CK,), jnp.float32),
          pltpu.VMEM((RPT,), jnp.float32),
      ],
  )
  def _sc_degree(dst_hbm, out_hbm, acc, dst_v, ones_v, buf_v):
    c = lax.axis_index("c")
    s = lax.axis_index("s")
    wid = s * NC + c
    pltpu.sync_copy(dst_hbm.at[wid], dst_v)

    one16 = jnp.ones((16,), jnp.float32)
    zero16 = jnp.zeros((16,), jnp.float32)

    @pl.loop(0, CK, step=16)
    def _(i):
      ones_v.at[pl.ds(i, 16)][...] = one16

    @pl.loop(0, RPT, step=16)
    def _(i):
      buf_v.at[pl.ds(i, 16)][...] = zero16

    pltpu.sync_copy(buf_v, acc.at[pl.ds(s * RPT, RPT)])
    plsc.subcore_barrier()

    @pl.loop(0, CH)
    def _(j):
      pltpu.sync_copy(ones_v, acc.at[dst_v.at[j]], add=True)

    plsc.subcore_barrier()
    pltpu.sync_copy(acc.at[pl.ds(s * RPT, RPT)], buf_v)
    pltpu.sync_copy(buf_v, out_hbm.at[c, pl.ds(s * RPT, RPT)])

  return _sc_degree


@functools.cache
def _sc_aggregate_kernel():
  mesh = plsc.VectorSubcoreMesh(core_axis_name="c", subcore_axis_name="s")

  @functools.partial(
      pl.kernel,
      out_type=jax.ShapeDtypeStruct((NC, NP, D), jnp.float32),
      mesh=mesh,
      scratch_types=[
          pltpu.VMEM_SHARED((NP, D), jnp.float32),
          pltpu.VMEM((CH, CK), jnp.int32),
          pltpu.VMEM((CH, CK), jnp.int32),
          pltpu.VMEM((CK, D), jnp.float32),
      ],
  )
  def _sc_aggregate(y_hbm, src_hbm, dst_hbm, out_hbm, acc, src_v, dst_v, rows_v):
    c = lax.axis_index("c")
    s = lax.axis_index("s")
    wid = s * NC + c
    pltpu.sync_copy(src_hbm.at[wid], src_v)
    pltpu.sync_copy(dst_hbm.at[wid], dst_v)

    zero16 = jnp.zeros((16,), jnp.float32)

    @pl.loop(0, CK)
    def _(r):
      @pl.loop(0, D, step=16)
      def _(q):
        rows_v.at[r, pl.ds(q, 16)][...] = zero16

    @pl.loop(0, RPT, step=CK)
    def _(i):
      pltpu.sync_copy(rows_v, acc.at[pl.ds(s * RPT + i, CK)])

    plsc.subcore_barrier()

    @pl.loop(0, CH)
    def _(j):
      pltpu.sync_copy(y_hbm.at[src_v.at[j]], rows_v)
      pltpu.sync_copy(rows_v, acc.at[dst_v.at[j]], add=True)

    plsc.subcore_barrier()

    @pl.loop(0, RPT, step=CK)
    def _(i):
      pltpu.sync_copy(acc.at[pl.ds(s * RPT + i, CK)], rows_v)
      pltpu.sync_copy(rows_v, out_hbm.at[c, pl.ds(s * RPT + i, CK)])

  return _sc_aggregate


def _tc_first_body(x_ref, w_ref, degp_ref, y_ref, dinv_ref):
    deg = degp_ref[0, 0:N, :] + degp_ref[1, 0:N, :] + 1.0
    dinv = lax.rsqrt(deg)
    xw = jnp.dot(x_ref[...], w_ref[...], preferred_element_type=jnp.float32)
    y_ref[0:N, :] = dinv * xw
    y_ref[N:NP, :] = jnp.zeros((NP - N, D), jnp.float32)
    dinv_ref[...] = dinv


def _tc_first(x, w1, degp):
    return pl.pallas_call(
        _tc_first_body,
        out_shape=(
            jax.ShapeDtypeStruct((NP, D), jnp.float32),
            jax.ShapeDtypeStruct((N, 1), jnp.float32),
        ),
    )(x, w1, degp)


def _bn_core(sp_ref, y_ref, dinv_ref, b_ref, g_ref, be_ref):
    s = sp_ref[0, 0:N, :] + sp_ref[1, 0:N, :]
    dinv = dinv_ref[...]
    t = dinv * (s + y_ref[0:N, :]) + b_ref[...]
    m = jnp.mean(t, axis=0, keepdims=True)
    v = jnp.mean(t * t, axis=0, keepdims=True) - m * m
    return g_ref[...] * (t - m) * lax.rsqrt(v + EPS) + be_ref[...]


def _tc_mid_body(sp_ref, y_ref, dinv_ref, b_ref, g_ref, be_ref, w_ref, yn_ref):
    h = jnp.maximum(_bn_core(sp_ref, y_ref, dinv_ref, b_ref, g_ref, be_ref), 0.0)
    xw = jnp.dot(h, w_ref[...], preferred_element_type=jnp.float32)
    yn_ref[0:N, :] = dinv_ref[...] * xw
    yn_ref[N:NP, :] = jnp.zeros((NP - N, D), jnp.float32)


def _tc_mid(sp, y, dinv, b, g, be, w_next):
    return pl.pallas_call(
        _tc_mid_body,
        out_shape=jax.ShapeDtypeStruct((NP, D), jnp.float32),
    )(sp, y, dinv, b.reshape(1, D), g.reshape(1, D), be.reshape(1, D), w_next)


def _tc_last_body(sp_ref, y_ref, dinv_ref, b_ref, g_ref, be_ref, o_ref):
    o_ref[...] = _bn_core(sp_ref, y_ref, dinv_ref, b_ref, g_ref, be_ref)


def _tc_last(sp, y, dinv, b, g, be):
    return pl.pallas_call(
        _tc_last_body,
        out_shape=jax.ShapeDtypeStruct((N, D), jnp.float32),
    )(sp, y, dinv, b.reshape(1, D), g.reshape(1, D), be.reshape(1, D))


def kernel(x, edge_index, W1, b1, g1, be1, W2, b2, g2, be2,
           W3, b3, g3, be3, W4, b4, g4, be4):
    ei = edge_index.astype(jnp.int32)
    npad = EPAD - E
    pad_idx = N + (jnp.arange(npad, dtype=jnp.int32) % (NP - N))
    src = jnp.concatenate([ei[0], pad_idx]).reshape(NW, CH, CK)
    dst = jnp.concatenate([ei[1], pad_idx]).reshape(NW, CH, CK)

    _sc_degree = _sc_degree_kernel()
    _sc_aggregate = _sc_aggregate_kernel()
    degp = _sc_degree(dst).reshape(NC, NP, 1)
    y, dinv = _tc_first(x, W1, degp)
    sp = _sc_aggregate(y, src, dst)
    y = _tc_mid(sp, y, dinv, b1, g1, be1, W2)
    sp = _sc_aggregate(y, src, dst)
    y = _tc_mid(sp, y, dinv, b2, g2, be2, W3)
    sp = _sc_aggregate(y, src, dst)
    y = _tc_mid(sp, y, dinv, b3, g3, be3, W4)
    sp = _sc_aggregate(y, src, dst)
    return _tc_last(sp, y, dinv, b4, g4, be4)


# speedup vs baseline: 21.4896x; 1.1987x over previous
import functools

import jax
import jax.numpy as jnp
from jax import lax
from jax.experimental import pallas as pl
from jax.experimental.pallas import tpu as pltpu
from jax.experimental.pallas import tpu_sc as plsc

N = 10000
E = 320000
D = 128
NP = 10240
NC = 2
NS = 16
NW = NC * NS
CK = 128
CH = 80
NBUF = 2
NIDX = 4
EW = CH * CK
EPAD = NW * EW
RPT = NP // NS
EPS = 1e-5

@functools.cache
def _sc_degree_kernel():
  mesh = plsc.VectorSubcoreMesh(core_axis_name="c", subcore_axis_name="s")

  @functools.partial(
      pl.kernel,
      out_type=jax.ShapeDtypeStruct((NC, NP), jnp.float32),
      mesh=mesh,
      scratch_types=[
          pltpu.VMEM_SHARED((NP,), jnp.float32),
          pltpu.VMEM((CH, CK), jnp.int32),
          pltpu.VMEM((CK,), jnp.float32),
          pltpu.VMEM((RPT,), jnp.float32),
      ],
  )
  def _sc_degree(dst_hbm, out_hbm, acc, dst_v, ones_v, buf_v):
    c = lax.axis_index("c")
    s = lax.axis_index("s")
    wid = s * NC + c
    pltpu.sync_copy(dst_hbm.at[wid], dst_v)

    one16 = jnp.ones((16,), jnp.float32)
    zero16 = jnp.zeros((16,), jnp.float32)

    @pl.loop(0, CK, step=16)
    def _(i):
      ones_v.at[pl.ds(i, 16)][...] = one16

    @pl.loop(0, RPT, step=16)
    def _(i):
      buf_v.at[pl.ds(i, 16)][...] = zero16

    pltpu.sync_copy(buf_v, acc.at[pl.ds(s * RPT, RPT)])
    plsc.subcore_barrier()

    @pl.loop(0, CH)
    def _(j):
      pltpu.sync_copy(ones_v, acc.at[dst_v.at[j]], add=True)

    plsc.subcore_barrier()
    pltpu.sync_copy(acc.at[pl.ds(s * RPT, RPT)], buf_v)
    pltpu.sync_copy(buf_v, out_hbm.at[c, pl.ds(s * RPT, RPT)])

  return _sc_degree


@functools.cache
def _sc_aggregate_kernel():
  mesh = plsc.VectorSubcoreMesh(core_axis_name="c", subcore_axis_name="s")

  @functools.partial(
      pl.kernel,
      out_type=jax.ShapeDtypeStruct((NC, NP, D), jnp.float32),
      mesh=mesh,
      scratch_types=[pltpu.VMEM_SHARED((NP, D), jnp.float32)]
      + [pltpu.VMEM((CK, D), jnp.float32) for _ in range(NBUF)]
      + [pltpu.VMEM((CK,), jnp.int32) for _ in range(2 * NIDX)]
      + [pltpu.SemaphoreType.DMA for _ in range(2 * NBUF + NIDX + 1)],
  )
  def _sc_aggregate(y_hbm, src_hbm, dst_hbm, out_hbm, acc, *scr):
    rows = scr[:NBUF]
    srcv = scr[NBUF:NBUF + NIDX]
    dstv = scr[NBUF + NIDX:NBUF + 2 * NIDX]
    base = NBUF + 2 * NIDX
    gsem = scr[base:base + NBUF]
    ssem = scr[base + NBUF:base + 2 * NBUF]
    isem = scr[base + 2 * NBUF:base + 2 * NBUF + NIDX]
    zsem = scr[-1]
    c = lax.axis_index("c")
    s = lax.axis_index("s")
    wid = s * NC + c

    zero16 = jnp.zeros((16,), jnp.float32)

    @pl.loop(0, CK)
    def _(r):
      @pl.loop(0, D, step=16)
      def _(q):
        rows[0].at[r, pl.ds(q, 16)][...] = zero16

    for i in range(RPT // CK):
      pltpu.async_copy(rows[0], acc.at[pl.ds(s * RPT + i * CK, CK)], zsem)

    for k in range(NIDX):
      pltpu.async_copy(src_hbm.at[wid, k], srcv[k], isem[k])
      pltpu.async_copy(dst_hbm.at[wid, k], dstv[k], isem[k])
    for i in range(RPT // CK):
      pltpu.make_async_copy(rows[0], acc.at[pl.ds(s * RPT, CK)], zsem).wait()
    for k in range(NBUF):
      for _ in range(2):
        pltpu.make_async_copy(src_hbm.at[wid, k], srcv[k], isem[k]).wait()

    for b in range(NBUF):
      pltpu.async_copy(y_hbm.at[srcv[b]], rows[b], gsem[b])

    plsc.subcore_barrier()

    @pl.loop(0, CH, step=2 * NBUF)
    def _(j):
      for t in (0, 1):
        b = t
        pltpu.make_async_copy(y_hbm.at[srcv[t]], rows[b], gsem[b]).wait()
        pltpu.async_copy(rows[b], acc.at[dstv[t]], ssem[b], add=True)
      for t in (0, 1):
        b = t
        pltpu.make_async_copy(rows[b], acc.at[dstv[t]], ssem[b]).wait()

        @pl.when(j + 4 + t < CH)
        def _():
          pltpu.async_copy(src_hbm.at[wid, j + 4 + t], srcv[t], isem[t])
          pltpu.async_copy(dst_hbm.at[wid, j + 4 + t], dstv[t], isem[t])

        for _ in range(2):
          pltpu.make_async_copy(src_hbm.at[wid, 0], srcv[t + 2],
                                isem[t + 2]).wait()
        pltpu.async_copy(y_hbm.at[srcv[t + 2]], rows[b], gsem[b])
      for t in (2, 3):
        b = t - 2
        pltpu.make_async_copy(y_hbm.at[srcv[t]], rows[b], gsem[b]).wait()
        pltpu.async_copy(rows[b], acc.at[dstv[t]], ssem[b], add=True)
      for t in (2, 3):
        b = t - 2
        pltpu.make_async_copy(rows[b], acc.at[dstv[t]], ssem[b]).wait()

        @pl.when(j + 4 + t < CH)
        def _():
          pltpu.async_copy(src_hbm.at[wid, j + 4 + t], srcv[t], isem[t])
          pltpu.async_copy(dst_hbm.at[wid, j + 4 + t], dstv[t], isem[t])

        @pl.when(j + 4 + b < CH)
        def _():
          for _ in range(2):
            pltpu.make_async_copy(src_hbm.at[wid, 0], srcv[b], isem[b]).wait()
          pltpu.async_copy(y_hbm.at[srcv[b]], rows[b], gsem[b])

    plsc.subcore_barrier()

    for i in range(RPT // CK):
      b = i % 2
      if i >= 2:
        pltpu.make_async_copy(rows[b], out_hbm.at[c, pl.ds(s * RPT, CK)],
                              gsem[b]).wait()
      pltpu.sync_copy(acc.at[pl.ds(s * RPT + i * CK, CK)], rows[b])
      pltpu.async_copy(rows[b], out_hbm.at[c, pl.ds(s * RPT + i * CK, CK)],
                       gsem[b])
    for b in range(2):
      pltpu.make_async_copy(rows[b], out_hbm.at[c, pl.ds(s * RPT, CK)],
                            gsem[b]).wait()

  return _sc_aggregate


def _tc_first_body(x_ref, w_ref, degp_ref, y_ref, dinv_ref):
    deg = degp_ref[0, 0:N, :] + degp_ref[1, 0:N, :] + 1.0
    dinv = lax.rsqrt(deg)
    xw = jnp.dot(x_ref[...], w_ref[...], preferred_element_type=jnp.float32)
    y_ref[0:N, :] = dinv * xw
    y_ref[N:NP, :] = jnp.zeros((NP - N, D), jnp.float32)
    dinv_ref[...] = dinv


def _tc_first(x, w1, degp):
    return pl.pallas_call(
        _tc_first_body,
        out_shape=(
            jax.ShapeDtypeStruct((NP, D), jnp.float32),
            jax.ShapeDtypeStruct((N, 1), jnp.float32),
        ),
    )(x, w1, degp)


def _bn_core(sp_ref, y_ref, dinv_ref, b_ref, g_ref, be_ref):
    s = sp_ref[0, 0:N, :] + sp_ref[1, 0:N, :]
    dinv = dinv_ref[...]
    t = dinv * (s + y_ref[0:N, :]) + b_ref[...]
    m = jnp.mean(t, axis=0, keepdims=True)
    v = jnp.mean(t * t, axis=0, keepdims=True) - m * m
    return g_ref[...] * (t - m) * lax.rsqrt(v + EPS) + be_ref[...]


def _tc_mid_body(sp_ref, y_ref, dinv_ref, b_ref, g_ref, be_ref, w_ref, yn_ref):
    h = jnp.maximum(_bn_core(sp_ref, y_ref, dinv_ref, b_ref, g_ref, be_ref), 0.0)
    xw = jnp.dot(h, w_ref[...], preferred_element_type=jnp.float32)
    yn_ref[0:N, :] = dinv_ref[...] * xw
    yn_ref[N:NP, :] = jnp.zeros((NP - N, D), jnp.float32)


def _tc_mid(sp, y, dinv, b, g, be, w_next):
    return pl.pallas_call(
        _tc_mid_body,
        out_shape=jax.ShapeDtypeStruct((NP, D), jnp.float32),
    )(sp, y, dinv, b.reshape(1, D), g.reshape(1, D), be.reshape(1, D), w_next)


def _tc_last_body(sp_ref, y_ref, dinv_ref, b_ref, g_ref, be_ref, o_ref):
    o_ref[...] = _bn_core(sp_ref, y_ref, dinv_ref, b_ref, g_ref, be_ref)


def _tc_last(sp, y, dinv, b, g, be):
    return pl.pallas_call(
        _tc_last_body,
        out_shape=jax.ShapeDtypeStruct((N, D), jnp.float32),
    )(sp, y, dinv, b.reshape(1, D), g.reshape(1, D), be.reshape(1, D))


def kernel(x, edge_index, W1, b1, g1, be1, W2, b2, g2, be2,
           W3, b3, g3, be3, W4, b4, g4, be4):
    ei = edge_index.astype(jnp.int32)
    npad = EPAD - E
    pad_idx = N + (jnp.arange(npad, dtype=jnp.int32) % (NP - N))
    src = jnp.concatenate([ei[0], pad_idx]).reshape(NW, CH, CK)
    dst = jnp.concatenate([ei[1], pad_idx]).reshape(NW, CH, CK)

    _sc_degree = _sc_degree_kernel()
    _sc_aggregate = _sc_aggregate_kernel()
    degp = _sc_degree(dst).reshape(NC, NP, 1)
    y, dinv = _tc_first(x, W1, degp)
    sp = _sc_aggregate(y, src, dst)
    y = _tc_mid(sp, y, dinv, b1, g1, be1, W2)
    sp = _sc_aggregate(y, src, dst)
    y = _tc_mid(sp, y, dinv, b2, g2, be2, W3)
    sp = _sc_aggregate(y, src, dst)
    y = _tc_mid(sp, y, dinv, b3, g3, be3, W4)
    sp = _sc_aggregate(y, src, dst)
    return _tc_last(sp, y, dinv, b4, g4, be4)


# speedup vs baseline: 23.2604x; 1.0824x over previous
import functools

import jax
import jax.numpy as jnp
from jax import lax
from jax.experimental import pallas as pl
from jax.experimental.pallas import tpu as pltpu
from jax.experimental.pallas import tpu_sc as plsc

N = 10000
E = 320000
D = 128
NP = 10240
NC = 2
NS = 16
NW = NC * NS
CK = 88
CH = 120
NBUF = 4
NIDX = 8
EW = CH * CK
EPAD = NW * EW
RPT = NP // NS
EPS = 1e-5

_PIECES = [(i * CK, CK) for i in range(RPT // CK)]
if RPT % CK:
  _PIECES.append((RPT - RPT % CK, RPT % CK))


@functools.cache
def _sc_degree_kernel():
  mesh = plsc.VectorSubcoreMesh(core_axis_name="c", subcore_axis_name="s")

  @functools.partial(
      pl.kernel,
      out_type=jax.ShapeDtypeStruct((NC, NP), jnp.float32),
      mesh=mesh,
      scratch_types=[
          pltpu.VMEM_SHARED((NP,), jnp.float32),
          pltpu.VMEM((2, CK), jnp.int32),
          pltpu.VMEM((96,), jnp.float32),
          pltpu.VMEM((RPT,), jnp.float32),
      ],
  )
  def _sc_degree(e_hbm, out_hbm, acc, idx_v, ones_v, buf_v):
    c = lax.axis_index("c")
    s = lax.axis_index("s")
    wid = s * NC + c

    one16 = jnp.ones((16,), jnp.float32)
    zero16 = jnp.zeros((16,), jnp.float32)

    @pl.loop(0, 96, step=16)
    def _(i):
      ones_v.at[pl.ds(i, 16)][...] = one16

    @pl.loop(0, RPT, step=16)
    def _(i):
      buf_v.at[pl.ds(i, 16)][...] = zero16

    pltpu.sync_copy(buf_v, acc.at[pl.ds(s * RPT, RPT)])
    plsc.subcore_barrier()

    @pl.loop(0, CH)
    def _(j):
      pltpu.sync_copy(e_hbm.at[wid, j], idx_v)
      pltpu.sync_copy(ones_v.at[pl.ds(0, CK)], acc.at[idx_v.at[1]], add=True)

    plsc.subcore_barrier()
    pltpu.sync_copy(acc.at[pl.ds(s * RPT, RPT)], buf_v)
    pltpu.sync_copy(buf_v, out_hbm.at[c, pl.ds(s * RPT, RPT)])

  return _sc_degree


@functools.cache
def _sc_aggregate_kernel():
  mesh = plsc.VectorSubcoreMesh(core_axis_name="c", subcore_axis_name="s")

  @functools.partial(
      pl.kernel,
      out_type=jax.ShapeDtypeStruct((NC, NP, D), jnp.float32),
      mesh=mesh,
      scratch_types=[pltpu.VMEM_SHARED((NP, D), jnp.float32)]
      + [pltpu.VMEM((CK, D), jnp.float32) for _ in range(NBUF)]
      + [pltpu.VMEM((2, CK), jnp.int32) for _ in range(NIDX)]
      + [pltpu.SemaphoreType.DMA for _ in range(2 * NBUF + NIDX + 1)],
  )
  def _sc_aggregate(y_hbm, e_hbm, out_hbm, acc, *scr):
    rows = scr[:NBUF]
    slots = scr[NBUF:NBUF + NIDX]
    gsem = scr[NBUF + NIDX:2 * NBUF + NIDX]
    ssem = scr[2 * NBUF + NIDX:3 * NBUF + NIDX]
    isem = scr[3 * NBUF + NIDX:3 * NBUF + 2 * NIDX]
    zsem = scr[-1]
    c = lax.axis_index("c")
    s = lax.axis_index("s")
    wid = s * NC + c

    zero16 = jnp.zeros((16,), jnp.float32)

    @pl.loop(0, CK)
    def _(r):
      @pl.loop(0, D, step=16)
      def _(q):
        rows[0].at[r, pl.ds(q, 16)][...] = zero16

    for off, ln in _PIECES:
      pltpu.async_copy(rows[0].at[pl.ds(0, ln)],
                       acc.at[pl.ds(s * RPT + off, ln)], zsem)

    for k in range(NIDX):
      pltpu.async_copy(e_hbm.at[wid, k], slots[k], isem[k])
    for off, ln in _PIECES:
      pltpu.make_async_copy(rows[0].at[pl.ds(0, ln)],
                            acc.at[pl.ds(s * RPT, ln)], zsem).wait()
    for k in range(NBUF):
      pltpu.make_async_copy(e_hbm.at[wid, k], slots[k], isem[k]).wait()

    for b in range(NBUF):
      pltpu.async_copy(y_hbm.at[slots[b].at[0]], rows[b], gsem[b])

    plsc.subcore_barrier()

    @pl.loop(0, CH, step=2 * NBUF)
    def _(j):
      for t in range(NBUF):
        b = t
        pltpu.make_async_copy(y_hbm.at[slots[t].at[0]], rows[b],
                              gsem[b]).wait()
        pltpu.async_copy(rows[b], acc.at[slots[t].at[1]], ssem[b], add=True)
      for t in range(NBUF):
        b = t
        pltpu.make_async_copy(rows[b], acc.at[slots[t].at[1]], ssem[b]).wait()

        @pl.when(j + NIDX + t < CH)
        def _():
          pltpu.async_copy(e_hbm.at[wid, j + NIDX + t], slots[t], isem[t])

        pltpu.make_async_copy(e_hbm.at[wid, 0], slots[t + NBUF],
                              isem[t + NBUF]).wait()
        pltpu.async_copy(y_hbm.at[slots[t + NBUF].at[0]], rows[b], gsem[b])
      for t in range(NBUF, 2 * NBUF):
        b = t - NBUF
        pltpu.make_async_copy(y_hbm.at[slots[t].at[0]], rows[b],
                              gsem[b]).wait()
        pltpu.async_copy(rows[b], acc.at[slots[t].at[1]], ssem[b], add=True)
      for t in range(NBUF, 2 * NBUF):
        b = t - NBUF
        pltpu.make_async_copy(rows[b], acc.at[slots[t].at[1]], ssem[b]).wait()

        @pl.when(j + NIDX + t < CH)
        def _():
          pltpu.async_copy(e_hbm.at[wid, j + NIDX + t], slots[t], isem[t])

        @pl.when(j + NIDX + b < CH)
        def _():
          pltpu.make_async_copy(e_hbm.at[wid, 0], slots[b], isem[b]).wait()
          pltpu.async_copy(y_hbm.at[slots[b].at[0]], rows[b], gsem[b])

    plsc.subcore_barrier()

    for i, (off, ln) in enumerate(_PIECES):
      b = i % 2
      if i >= 2:
        po, pl_ = _PIECES[i - 2]
        pltpu.make_async_copy(rows[b].at[pl.ds(0, pl_)],
                              out_hbm.at[c, pl.ds(s * RPT, pl_)],
                              gsem[b]).wait()
      pltpu.sync_copy(acc.at[pl.ds(s * RPT + off, ln)],
                      rows[b].at[pl.ds(0, ln)])
      pltpu.async_copy(rows[b].at[pl.ds(0, ln)],
                       out_hbm.at[c, pl.ds(s * RPT + off, ln)], gsem[b])
    for i in range(max(0, len(_PIECES) - 2), len(_PIECES)):
      b = i % 2
      off, ln = _PIECES[i]
      pltpu.make_async_copy(rows[b].at[pl.ds(0, ln)],
                            out_hbm.at[c, pl.ds(s * RPT, ln)],
                            gsem[b]).wait()

  return _sc_aggregate


def _tc_first_body(x_ref, w_ref, degp_ref, y_ref, dinv_ref):
    deg = degp_ref[0, 0:N, :] + degp_ref[1, 0:N, :] + 1.0
    dinv = lax.rsqrt(deg)
    xw = jnp.dot(x_ref[...], w_ref[...], preferred_element_type=jnp.float32)
    y_ref[0:N, :] = dinv * xw
    y_ref[N:NP, :] = jnp.zeros((NP - N, D), jnp.float32)
    dinv_ref[...] = dinv


def _tc_first(x, w1, degp):
    return pl.pallas_call(
        _tc_first_body,
        out_shape=(
            jax.ShapeDtypeStruct((NP, D), jnp.float32),
            jax.ShapeDtypeStruct((N, 1), jnp.float32),
        ),
    )(x, w1, degp)


def _bn_core(sp_ref, y_ref, dinv_ref, b_ref, g_ref, be_ref):
    s = sp_ref[0, 0:N, :] + sp_ref[1, 0:N, :]
    dinv = dinv_ref[...]
    t = dinv * (s + y_ref[0:N, :]) + b_ref[...]
    m = jnp.mean(t, axis=0, keepdims=True)
    v = jnp.mean(t * t, axis=0, keepdims=True) - m * m
    return g_ref[...] * (t - m) * lax.rsqrt(v + EPS) + be_ref[...]


def _tc_mid_body(sp_ref, y_ref, dinv_ref, b_ref, g_ref, be_ref, w_ref, yn_ref):
    h = jnp.maximum(_bn_core(sp_ref, y_ref, dinv_ref, b_ref, g_ref, be_ref), 0.0)
    xw = jnp.dot(h, w_ref[...], preferred_element_type=jnp.float32)
    yn_ref[0:N, :] = dinv_ref[...] * xw
    yn_ref[N:NP, :] = jnp.zeros((NP - N, D), jnp.float32)


def _tc_mid(sp, y, dinv, b, g, be, w_next):
    return pl.pallas_call(
        _tc_mid_body,
        out_shape=jax.ShapeDtypeStruct((NP, D), jnp.float32),
    )(sp, y, dinv, b.reshape(1, D), g.reshape(1, D), be.reshape(1, D), w_next)


def _tc_last_body(sp_ref, y_ref, dinv_ref, b_ref, g_ref, be_ref, o_ref):
    o_ref[...] = _bn_core(sp_ref, y_ref, dinv_ref, b_ref, g_ref, be_ref)


def _tc_last(sp, y, dinv, b, g, be):
    return pl.pallas_call(
        _tc_last_body,
        out_shape=jax.ShapeDtypeStruct((N, D), jnp.float32),
    )(sp, y, dinv, b.reshape(1, D), g.reshape(1, D), be.reshape(1, D))


def kernel(x, edge_index, W1, b1, g1, be1, W2, b2, g2, be2,
           W3, b3, g3, be3, W4, b4, g4, be4):
    ei = edge_index.astype(jnp.int32)
    npad = EPAD - E
    pad_idx = N + (jnp.arange(npad, dtype=jnp.int32) % (NP - N))
    src = jnp.concatenate([ei[0], pad_idx])
    dst = jnp.concatenate([ei[1], pad_idx])
    edges = jnp.stack(
        [src.reshape(NW, CH, CK), dst.reshape(NW, CH, CK)], axis=2)

    _sc_degree = _sc_degree_kernel()
    _sc_aggregate = _sc_aggregate_kernel()
    degp = _sc_degree(edges).reshape(NC, NP, 1)
    y, dinv = _tc_first(x, W1, degp)
    sp = _sc_aggregate(y, edges)
    y = _tc_mid(sp, y, dinv, b1, g1, be1, W2)
    sp = _sc_aggregate(y, edges)
    y = _tc_mid(sp, y, dinv, b2, g2, be2, W3)
    sp = _sc_aggregate(y, edges)
    y = _tc_mid(sp, y, dinv, b3, g3, be3, W4)
    sp = _sc_aggregate(y, edges)
    return _tc_last(sp, y, dinv, b4, g4, be4)


# speedup vs baseline: 25.9569x; 1.1159x over previous
import functools

import jax
import jax.numpy as jnp
from jax import lax
from jax.experimental import pallas as pl
from jax.experimental.pallas import tpu as pltpu
from jax.experimental.pallas import tpu_sc as plsc

N = 10000
E = 320000
D = 128
NP = 10240
NC = 2
NS = 16
NW = NC * NS
CK = 88
CH = 120
NBUF = 4
NIDX = 8
EW = CH * CK
EPAD = NW * EW
RPT = NP // NS
EPS = 1e-5

_PIECES = [(i * CK, CK) for i in range(RPT // CK)]
if RPT % CK:
  _PIECES.append((RPT - RPT % CK, RPT % CK))


@functools.cache
def _sc_degree_kernel():
  mesh = plsc.VectorSubcoreMesh(core_axis_name="c", subcore_axis_name="s")

  @functools.partial(
      pl.kernel,
      out_type=jax.ShapeDtypeStruct((NC, NP), jnp.float32),
      mesh=mesh,
      scratch_types=[
          pltpu.VMEM_SHARED((NP,), jnp.float32),
          pltpu.VMEM((CH, 2, CK), jnp.int32),
          pltpu.VMEM((96,), jnp.float32),
          pltpu.VMEM((RPT,), jnp.float32),
          pltpu.SemaphoreType.DMA,
      ]
      + [pltpu.SemaphoreType.DMA for _ in range(8)],
  )
  def _sc_degree(e_hbm, out_hbm, acc, idx_v, ones_v, buf_v, isem, *ssem):
    c = lax.axis_index("c")
    s = lax.axis_index("s")
    wid = s * NC + c
    pltpu.async_copy(e_hbm.at[wid], idx_v, isem)

    one16 = jnp.ones((16,), jnp.float32)
    zero16 = jnp.zeros((16,), jnp.float32)

    @pl.loop(0, 96, step=16)
    def _(i):
      ones_v.at[pl.ds(i, 16)][...] = one16

    @pl.loop(0, RPT, step=16)
    def _(i):
      buf_v.at[pl.ds(i, 16)][...] = zero16

    pltpu.sync_copy(buf_v, acc.at[pl.ds(s * RPT, RPT)])
    pltpu.make_async_copy(e_hbm.at[wid], idx_v, isem).wait()
    plsc.subcore_barrier()

    ones = ones_v.at[pl.ds(0, CK)]

    @pl.loop(0, CH, step=8)
    def _(j):
      for t in range(8):
        @pl.when(j > 0)
        def _():
          pltpu.make_async_copy(ones, acc.at[pl.ds(0, CK)], ssem[t]).wait()

        pltpu.async_copy(ones, acc.at[idx_v.at[j + t, 1]], ssem[t], add=True)

    for t in range(8):
      pltpu.make_async_copy(ones, acc.at[pl.ds(0, CK)], ssem[t]).wait()

    plsc.subcore_barrier()
    pltpu.sync_copy(acc.at[pl.ds(s * RPT, RPT)], buf_v)
    pltpu.sync_copy(buf_v, out_hbm.at[c, pl.ds(s * RPT, RPT)])

  return _sc_degree


@functools.cache
def _sc_aggregate_kernel():
  mesh = plsc.VectorSubcoreMesh(core_axis_name="c", subcore_axis_name="s")

  @functools.partial(
      pl.kernel,
      out_type=jax.ShapeDtypeStruct((NC, NP, D), jnp.float32),
      mesh=mesh,
      scratch_types=[pltpu.VMEM_SHARED((NP, D), jnp.float32)]
      + [pltpu.VMEM((CK, D), jnp.float32) for _ in range(NBUF)]
      + [pltpu.VMEM((2, CK), jnp.int32) for _ in range(NIDX)]
      + [pltpu.SemaphoreType.DMA for _ in range(2 * NBUF + NIDX + 1)],
  )
  def _sc_aggregate(y_hbm, e_hbm, out_hbm, acc, *scr):
    rows = scr[:NBUF]
    slots = scr[NBUF:NBUF + NIDX]
    gsem = scr[NBUF + NIDX:2 * NBUF + NIDX]
    ssem = scr[2 * NBUF + NIDX:3 * NBUF + NIDX]
    isem = scr[3 * NBUF + NIDX:3 * NBUF + 2 * NIDX]
    zsem = scr[-1]
    c = lax.axis_index("c")
    s = lax.axis_index("s")
    wid = s * NC + c

    zero16 = jnp.zeros((16,), jnp.float32)

    @pl.loop(0, CK)
    def _(r):
      @pl.loop(0, D, step=16)
      def _(q):
        rows[0].at[r, pl.ds(q, 16)][...] = zero16

    for off, ln in _PIECES:
      pltpu.async_copy(rows[0].at[pl.ds(0, ln)],
                       acc.at[pl.ds(s * RPT + off, ln)], zsem)

    for k in range(NIDX):
      pltpu.async_copy(e_hbm.at[wid, k], slots[k], isem[k])
    for off, ln in _PIECES:
      pltpu.make_async_copy(rows[0].at[pl.ds(0, ln)],
                            acc.at[pl.ds(s * RPT, ln)], zsem).wait()
    for k in range(NBUF):
      pltpu.make_async_copy(e_hbm.at[wid, k], slots[k], isem[k]).wait()

    for b in range(NBUF):
      pltpu.async_copy(y_hbm.at[slots[b].at[0]], rows[b], gsem[b])

    plsc.subcore_barrier()

    @pl.loop(0, CH, step=2 * NBUF)
    def _(j):
      for t in range(NBUF):
        b = t
        pltpu.make_async_copy(y_hbm.at[slots[t].at[0]], rows[b],
                              gsem[b]).wait()
        pltpu.async_copy(rows[b], acc.at[slots[t].at[1]], ssem[b], add=True)
      for t in range(NBUF):
        b = t
        pltpu.make_async_copy(rows[b], acc.at[slots[t].at[1]], ssem[b]).wait()

        @pl.when(j + NIDX + t < CH)
        def _():
          pltpu.async_copy(e_hbm.at[wid, j + NIDX + t], slots[t], isem[t])

        pltpu.make_async_copy(e_hbm.at[wid, 0], slots[t + NBUF],
                              isem[t + NBUF]).wait()
        pltpu.async_copy(y_hbm.at[slots[t + NBUF].at[0]], rows[b], gsem[b])
      for t in range(NBUF, 2 * NBUF):
        b = t - NBUF
        pltpu.make_async_copy(y_hbm.at[slots[t].at[0]], rows[b],
                              gsem[b]).wait()
        pltpu.async_copy(rows[b], acc.at[slots[t].at[1]], ssem[b], add=True)
      for t in range(NBUF, 2 * NBUF):
        b = t - NBUF
        pltpu.make_async_copy(rows[b], acc.at[slots[t].at[1]], ssem[b]).wait()

        @pl.when(j + NIDX + t < CH)
        def _():
          pltpu.async_copy(e_hbm.at[wid, j + NIDX + t], slots[t], isem[t])

        @pl.when(j + NIDX + b < CH)
        def _():
          pltpu.make_async_copy(e_hbm.at[wid, 0], slots[b], isem[b]).wait()
          pltpu.async_copy(y_hbm.at[slots[b].at[0]], rows[b], gsem[b])

    plsc.subcore_barrier()

    for i, (off, ln) in enumerate(_PIECES):
      b = i % 2
      if i >= 2:
        po, pl_ = _PIECES[i - 2]
        pltpu.make_async_copy(rows[b].at[pl.ds(0, pl_)],
                              out_hbm.at[c, pl.ds(s * RPT, pl_)],
                              gsem[b]).wait()
      pltpu.sync_copy(acc.at[pl.ds(s * RPT + off, ln)],
                      rows[b].at[pl.ds(0, ln)])
      pltpu.async_copy(rows[b].at[pl.ds(0, ln)],
                       out_hbm.at[c, pl.ds(s * RPT + off, ln)], gsem[b])
    for i in range(max(0, len(_PIECES) - 2), len(_PIECES)):
      b = i % 2
      off, ln = _PIECES[i]
      pltpu.make_async_copy(rows[b].at[pl.ds(0, ln)],
                            out_hbm.at[c, pl.ds(s * RPT, ln)],
                            gsem[b]).wait()

  return _sc_aggregate


def _tc_first_body(x_ref, w_ref, degp_ref, y_ref, dinv_ref):
    deg = degp_ref[0, 0:N, :] + degp_ref[1, 0:N, :] + 1.0
    dinv = lax.rsqrt(deg)
    xw = jnp.dot(x_ref[...], w_ref[...], preferred_element_type=jnp.float32)
    y_ref[0:N, :] = dinv * xw
    y_ref[N:NP, :] = jnp.zeros((NP - N, D), jnp.float32)
    dinv_ref[...] = dinv


def _tc_first(x, w1, degp):
    return pl.pallas_call(
        _tc_first_body,
        out_shape=(
            jax.ShapeDtypeStruct((NP, D), jnp.float32),
            jax.ShapeDtypeStruct((N, 1), jnp.float32),
        ),
    )(x, w1, degp)


def _bn_core(sp_ref, y_ref, dinv_ref, b_ref, g_ref, be_ref):
    s = sp_ref[0, 0:N, :] + sp_ref[1, 0:N, :]
    dinv = dinv_ref[...]
    t = dinv * (s + y_ref[0:N, :]) + b_ref[...]
    m = jnp.mean(t, axis=0, keepdims=True)
    v = jnp.mean(t * t, axis=0, keepdims=True) - m * m
    return g_ref[...] * (t - m) * lax.rsqrt(v + EPS) + be_ref[...]


def _tc_mid_body(sp_ref, y_ref, dinv_ref, b_ref, g_ref, be_ref, w_ref, yn_ref):
    h = jnp.maximum(_bn_core(sp_ref, y_ref, dinv_ref, b_ref, g_ref, be_ref), 0.0)
    xw = jnp.dot(h, w_ref[...], preferred_element_type=jnp.float32)
    yn_ref[0:N, :] = dinv_ref[...] * xw
    yn_ref[N:NP, :] = jnp.zeros((NP - N, D), jnp.float32)


def _tc_mid(sp, y, dinv, b, g, be, w_next):
    return pl.pallas_call(
        _tc_mid_body,
        out_shape=jax.ShapeDtypeStruct((NP, D), jnp.float32),
    )(sp, y, dinv, b.reshape(1, D), g.reshape(1, D), be.reshape(1, D), w_next)


def _tc_last_body(sp_ref, y_ref, dinv_ref, b_ref, g_ref, be_ref, o_ref):
    o_ref[...] = _bn_core(sp_ref, y_ref, dinv_ref, b_ref, g_ref, be_ref)


def _tc_last(sp, y, dinv, b, g, be):
    return pl.pallas_call(
        _tc_last_body,
        out_shape=jax.ShapeDtypeStruct((N, D), jnp.float32),
    )(sp, y, dinv, b.reshape(1, D), g.reshape(1, D), be.reshape(1, D))


def kernel(x, edge_index, W1, b1, g1, be1, W2, b2, g2, be2,
           W3, b3, g3, be3, W4, b4, g4, be4):
    ei = edge_index.astype(jnp.int32)
    npad = EPAD - E
    pad_idx = N + (jnp.arange(npad, dtype=jnp.int32) % (NP - N))
    src = jnp.concatenate([ei[0], pad_idx])
    dst = jnp.concatenate([ei[1], pad_idx])
    edges = jnp.stack(
        [src.reshape(NW, CH, CK), dst.reshape(NW, CH, CK)], axis=2)

    _sc_degree = _sc_degree_kernel()
    _sc_aggregate = _sc_aggregate_kernel()
    degp = _sc_degree(edges).reshape(NC, NP, 1)
    y, dinv = _tc_first(x, W1, degp)
    sp = _sc_aggregate(y, edges)
    y = _tc_mid(sp, y, dinv, b1, g1, be1, W2)
    sp = _sc_aggregate(y, edges)
    y = _tc_mid(sp, y, dinv, b2, g2, be2, W3)
    sp = _sc_aggregate(y, edges)
    y = _tc_mid(sp, y, dinv, b3, g3, be3, W4)
    sp = _sc_aggregate(y, edges)
    return _tc_last(sp, y, dinv, b4, g4, be4)
